# baseline (speedup 1.0000x reference)
import functools

import jax
import jax.numpy as jnp
from jax import lax
from jax.experimental import pallas as pl
from jax.experimental.pallas import tpu as pltpu
from jax.experimental.pallas import tpu_sc as plsc

B = 1024
L = 50
BL = B * L
G = 10
D = 32
NF = 100000
ZROW = NF

NC = 2
NS = 16
NW = NC * NS
LANES = 16

NBT = B // 128
NUNIT = L * NBT * 2
UPW = NUNIT // NW
GPU = G // 2
TILES = GPU * 4
SROWS = GPU * 128
LSTRIDE = (G * D // 8) * 8192

_mesh = plsc.VectorSubcoreMesh(
    core_axis_name="c", subcore_axis_name="s", num_cores=NC, num_subcores=NS
)


@functools.partial(
    pl.kernel,
    out_type=jax.ShapeDtypeStruct((B * L * G * D,), jnp.float32),
    mesh=_mesh,
    compiler_params=pltpu.CompilerParams(
        use_tc_tiling_on_sc=False, needs_layout_passes=False),
    scratch_types=[
        pltpu.VMEM((1, 128), jnp.int32),
        pltpu.VMEM((128, 80), jnp.int32),
        pltpu.VMEM((128,), jnp.float32),
        pltpu.VMEM((GPU, 128), jnp.int32),
        pltpu.VMEM((GPU, 128), jnp.int32),
        pltpu.VMEM((SROWS, D), jnp.float32),
        pltpu.VMEM((SROWS, D), jnp.float32),
        pltpu.VMEM((TILES * 1024,), jnp.float32),
        pltpu.VMEM((TILES * 1024,), jnp.float32),
        pltpu.SemaphoreType.DMA,
        pltpu.SemaphoreType.DMA,
        pltpu.SemaphoreType.DMA,
        pltpu.SemaphoreType.DMA,
        pltpu.SemaphoreType.DMA,
    ],
)
def _feat_embed(sel_hbm, mask_hbm, table_hbm, out_hbm,
                idxsel, selbuf, maskv, idxt0, idxt1, rows0, rows1,
                stripe0, stripe1, semsel, semg0, semg1, semo0, semo1):
    idxt = (idxt0, idxt1)
    rows = (rows0, rows1)
    stripe = (stripe0, stripe1)
    semg = (semg0, semg1)
    semo = (semo0, semo1)

    wid = lax.axis_index("s") * NC + lax.axis_index("c")
    iota = lax.iota(jnp.int32, LANES)
    pats = [[(4 * gg + 2 * v) * 1024
             + (iota >= 8).astype(jnp.int32) * 1024
             + (iota & 7) * 128
             for v in (0, 1)] for gg in range(GPU)]

    def unit_params(u):
        uu = wid * UPW + u
        l = uu >> 4
        bt = (uu >> 1) & 7
        h = uu & 1
        return l, bt, h

    def stage(u, b):
        l, bt, h = unit_params(u)
        K = bt * (128 * L) + l
        for j in range(8):
            i_vec = j * LANES + iota
            idxsel[0, pl.ds(j * LANES, LANES)] = (L * i_vec + K) >> 3
        ovec = ((2 * iota + K) & 7) * G
        pltpu.async_copy(sel_hbm.at[idxsel.at[0]], selbuf, semsel).wait()
        pltpu.sync_copy(
            mask_hbm.at[pl.ds(pl.multiple_of(l * B + bt * 128, 8), 128)],
            maskv)
        for gg in range(GPU):
            col = ovec + (h * GPU + gg)
            for j in range(8):
                pos = j * LANES + iota
                sv = plsc.load_gather(selbuf, [pos, col])
                mv = maskv[pl.ds(j * LANES, LANES)]
                idxt[b][gg, pl.ds(j * LANES, LANES)] = jnp.where(
                    mv > 0.5, sv, ZROW)
        for gg in range(GPU):
            pltpu.async_copy(
                table_hbm.at[idxt[b].at[gg]],
                rows[b].at[pl.ds(gg * 128, 128)],
                semg[b])

    def wait_main(b):
        for gg in range(GPU):
            pltpu.make_async_copy(
                table_hbm.at[idxt[b].at[gg]],
                rows[b].at[pl.ds(gg * 128, 128)],
                semg[b]).wait()

    def assemble(b):
        def pos_body(p, carry):
            bp = lax.broadcast(p, (LANES,))
            for gg in range(GPU):
                row = gg * 128 + p
                for v in (0, 1):
                    x = rows[b][row, pl.ds(16 * v, 16)]
                    plsc.store_scatter(stripe[b], [pats[gg][v] + bp], x)
            return carry
        lax.fori_loop(0, 128, pos_body, 0, unroll=False)

    def out_addr(u, t):
        l, bt, h = unit_params(u)
        return pl.multiple_of(
            l * LSTRIDE + (h * TILES + t) * 8192 + bt * 1024, 8)

    def fire_outs(u, b):
        for t in range(TILES):
            pltpu.async_copy(
                stripe[b].at[pl.ds(t * 1024, 1024)],
                out_hbm.at[pl.ds(out_addr(u, t), 1024)],
                semo[b])

    def drain_outs(u, b):
        for t in range(TILES):
            pltpu.make_async_copy(
                stripe[b].at[pl.ds(t * 1024, 1024)],
                out_hbm.at[pl.ds(out_addr(u, t), 1024)],
                semo[b]).wait()

    stage(0, 0)

    def pair_body(cc, carry):
        for b in (0, 1):
            u = cc * 2 + b

            @pl.when(u + 1 < UPW)
            def _fire_next():
                @pl.when(u >= 1)
                def _drain_prev():
                    drain_outs(u - 1, 1 - b)
                stage(u + 1, 1 - b)

            wait_main(b)
            assemble(b)
            fire_outs(u, b)
        return carry

    lax.fori_loop(0, (UPW - 1) // 2, pair_body, 0, unroll=False)
    wait_main(0)
    assemble(0)
    fire_outs(UPW - 1, 0)
    drain_outs(UPW - 2, 1)
    drain_outs(UPW - 1, 0)


def kernel(feat_matrix, padding, embed_table):
    sel = feat_matrix[:, :, :G].reshape(BL * G // 80, 80).astype(jnp.int32)
    maskt = jnp.transpose(
        1.0 - padding.astype(jnp.float32)).reshape(-1)
    table2 = jnp.concatenate(
        [embed_table, jnp.zeros((8, D), jnp.float32)], axis=0)
    out = _feat_embed(sel, maskt, table2)
    o = out.reshape(L, G * D // 8, B // 128, 8, 128)
    o = o.transpose(2, 4, 0, 1, 3)
    return o.reshape(B, L, G * D)


# speedup vs baseline: 1.0032x; 1.0032x over previous
import functools

import jax
import jax.numpy as jnp
from jax import lax
from jax.experimental import pallas as pl
from jax.experimental.pallas import tpu as pltpu
from jax.experimental.pallas import tpu_sc as plsc

B = 1024
L = 50
BL = B * L
G = 10
D = 32
NF = 100000
ZROW = NF

NC = 2
NS = 16
NW = NC * NS
LANES = 16

NBT = B // 128
NUNIT = L * NBT * 2
UPW = NUNIT // NW
GPU = G // 2
TILES = GPU * 4
SROWS = GPU * 128
LSTRIDE = (G * D // 8) * 8192

_mesh = plsc.VectorSubcoreMesh(
    core_axis_name="c", subcore_axis_name="s", num_cores=NC, num_subcores=NS
)


@functools.partial(
    pl.kernel,
    out_type=jax.ShapeDtypeStruct((B * L * G * D,), jnp.float32),
    mesh=_mesh,
    compiler_params=pltpu.CompilerParams(
        use_tc_tiling_on_sc=False, needs_layout_passes=False),
    scratch_types=[
        pltpu.VMEM((1, 128), jnp.int32),
        pltpu.VMEM((128, 80), jnp.int32),
        pltpu.VMEM((128,), jnp.float32),
        pltpu.VMEM((GPU, 128), jnp.int32),
        pltpu.VMEM((GPU, 128), jnp.int32),
        pltpu.VMEM((SROWS, D), jnp.float32),
        pltpu.VMEM((SROWS, D), jnp.float32),
        pltpu.VMEM((TILES * 1024,), jnp.float32),
        pltpu.VMEM((TILES * 1024,), jnp.float32),
        pltpu.SemaphoreType.DMA,
        pltpu.SemaphoreType.DMA,
        pltpu.SemaphoreType.DMA,
        pltpu.SemaphoreType.DMA,
        pltpu.SemaphoreType.DMA,
    ],
)
def _feat_embed(sel_hbm, mask_hbm, table_hbm, out_hbm,
                idxsel, selbuf, maskv, idxt0, idxt1, rows0, rows1,
                stripe0, stripe1, semsel, semg0, semg1, semo0, semo1):
    idxt = (idxt0, idxt1)
    rows = (rows0, rows1)
    stripe = (stripe0, stripe1)
    semg = (semg0, semg1)
    semo = (semo0, semo1)

    wid = lax.axis_index("s") * NC + lax.axis_index("c")
    iota = lax.iota(jnp.int32, LANES)
    pats = [[(4 * gg + 2 * v) * 1024
             + (iota >= 8).astype(jnp.int32) * 1024
             + (iota & 7) * 128
             for v in (0, 1)] for gg in range(GPU)]

    def unit_params(u):
        uu = wid * UPW + u
        l = uu >> 4
        bt = (uu >> 1) & 7
        h = uu & 1
        return l, bt, h

    def stage(u, b):
        l, bt, h = unit_params(u)
        K = bt * (128 * L) + l
        for j in range(8):
            i_vec = j * LANES + iota
            idxsel[0, pl.ds(j * LANES, LANES)] = (L * i_vec + K) >> 3
        ovec = ((2 * iota + K) & 7) * G
        pltpu.async_copy(sel_hbm.at[idxsel.at[0]], selbuf, semsel).wait()
        pltpu.sync_copy(
            mask_hbm.at[pl.ds(pl.multiple_of(l * B + bt * 128, 8), 128)],
            maskv)
        for gg in range(GPU):
            col = ovec + (h * GPU + gg)
            for j in range(8):
                pos = j * LANES + iota
                sv = plsc.load_gather(selbuf, [pos, col])
                mv = maskv[pl.ds(j * LANES, LANES)]
                idxt[b][gg, pl.ds(j * LANES, LANES)] = jnp.where(
                    mv > 0.5, sv, ZROW)
        for gg in range(GPU):
            pltpu.async_copy(
                table_hbm.at[idxt[b].at[gg]],
                rows[b].at[pl.ds(gg * 128, 128)],
                semg[b])

    def wait_main(b):
        for gg in range(GPU):
            pltpu.make_async_copy(
                table_hbm.at[idxt[b].at[gg]],
                rows[b].at[pl.ds(gg * 128, 128)],
                semg[b]).wait()

    def assemble(b):
        def pb_body(pb, carry):
            def e_body(e0, c2):
                rot = (e0 + iota) & 31
                dpat = ((rot >> 3) << 10) + ((rot & 7) << 7) + iota
                src_row0 = iota + pb * 16
                for gg in range(GPU):
                    x = plsc.load_gather(
                        rows[b], [src_row0 + gg * 128, rot])
                    plsc.store_scatter(
                        stripe[b], [dpat + (pb * 16 + gg * 4096)], x)
                return c2
            lax.fori_loop(0, 32, e_body, 0, unroll=False)
            return carry
        lax.fori_loop(0, 8, pb_body, 0, unroll=False)

    def out_addr(u, t):
        l, bt, h = unit_params(u)
        return pl.multiple_of(
            l * LSTRIDE + (h * TILES + t) * 8192 + bt * 1024, 8)

    def fire_outs(u, b):
        for t in range(TILES):
            pltpu.async_copy(
                stripe[b].at[pl.ds(t * 1024, 1024)],
                out_hbm.at[pl.ds(out_addr(u, t), 1024)],
                semo[b])

    def drain_outs(u, b):
        for t in range(TILES):
            pltpu.make_async_copy(
                stripe[b].at[pl.ds(t * 1024, 1024)],
                out_hbm.at[pl.ds(out_addr(u, t), 1024)],
                semo[b]).wait()

    stage(0, 0)

    def pair_body(cc, carry):
        for b in (0, 1):
            u = cc * 2 + b

            @pl.when(u + 1 < UPW)
            def _fire_next():
                @pl.when(u >= 1)
                def _drain_prev():
                    drain_outs(u - 1, 1 - b)
                stage(u + 1, 1 - b)

            wait_main(b)
            assemble(b)
            fire_outs(u, b)
        return carry

    lax.fori_loop(0, (UPW - 1) // 2, pair_body, 0, unroll=False)
    wait_main(0)
    assemble(0)
    fire_outs(UPW - 1, 0)
    drain_outs(UPW - 2, 1)
    drain_outs(UPW - 1, 0)


def kernel(feat_matrix, padding, embed_table):
    sel = feat_matrix[:, :, :G].reshape(BL * G // 80, 80).astype(jnp.int32)
    maskt = jnp.transpose(
        1.0 - padding.astype(jnp.float32)).reshape(-1)
    table2 = jnp.concatenate(
        [embed_table, jnp.zeros((8, D), jnp.float32)], axis=0)
    out = _feat_embed(sel, maskt, table2)
    o = out.reshape(L, G * D // 8, B // 128, 8, 128)
    o = o.transpose(2, 4, 0, 1, 3)
    return o.reshape(B, L, G * D)


# speedup vs baseline: 8.7160x; 8.6879x over previous
import functools

import jax
import jax.numpy as jnp
from jax import lax
from jax.experimental import pallas as pl
from jax.experimental.pallas import tpu as pltpu
from jax.experimental.pallas import tpu_sc as plsc

B = 1024
L = 50
BL = B * L
G = 10
D = 32

NC = 2
NS = 16
NW = NC * NS
LANES = 16

POS_PER_W = BL // NW
CHUNK = 80
NCHUNK = POS_PER_W // CHUNK
STREAM = 80
NGRP = CHUNK * G // STREAM
ROWS = CHUNK * G

_mesh = plsc.VectorSubcoreMesh(
    core_axis_name="c", subcore_axis_name="s", num_cores=NC, num_subcores=NS
)


@functools.partial(
    pl.kernel,
    out_type=jax.ShapeDtypeStruct((BL * G, D), jnp.float32),
    mesh=_mesh,
    compiler_params=pltpu.CompilerParams(
        use_tc_tiling_on_sc=False, needs_layout_passes=False),
    scratch_types=[
        pltpu.VMEM((NGRP, STREAM), jnp.int32),
        pltpu.VMEM((NGRP, STREAM), jnp.int32),
        pltpu.VMEM((CHUNK,), jnp.float32),
        pltpu.VMEM((CHUNK,), jnp.float32),
        pltpu.VMEM((ROWS, D), jnp.float32),
        pltpu.VMEM((ROWS, D), jnp.float32),
        pltpu.SemaphoreType.DMA,
        pltpu.SemaphoreType.DMA,
        pltpu.SemaphoreType.DMA,
        pltpu.SemaphoreType.DMA,
    ],
)
def _feat_embed(sel_hbm, mask_hbm, table_hbm, out_hbm,
                idx0, idx1, mask0, mask1, rows0, rows1,
                semg0, semg1, semo0, semo1):
    idx = (idx0, idx1)
    maskv = (mask0, mask1)
    rows = (rows0, rows1)
    semg = (semg0, semg1)
    semo = (semo0, semo1)

    wid = lax.axis_index("s") * NC + lax.axis_index("c")
    wpos0 = wid * POS_PER_W

    def stage_and_fire(c, b):
        pos0 = pl.multiple_of(wpos0 + c * CHUNK, 8)
        pltpu.sync_copy(sel_hbm.at[pl.ds(pos0 * G // STREAM, NGRP)], idx[b])
        pltpu.sync_copy(mask_hbm.at[pl.ds(pos0, CHUNK)], maskv[b])
        for g in range(NGRP):
            pltpu.async_copy(
                table_hbm.at[idx[b].at[g]],
                rows[b].at[pl.ds(g * STREAM, STREAM)],
                semg[b])

    def wait_gathers(b):
        for g in range(NGRP):
            pltpu.make_async_copy(
                table_hbm.at[idx[b].at[g]],
                rows[b].at[pl.ds(g * STREAM, STREAM)],
                semg[b]).wait()

    def out_slice(c):
        row0 = pl.multiple_of((wpos0 + c * CHUNK) * G, 8)
        return out_hbm.at[pl.ds(row0, ROWS)]

    def drain_out(c, b):
        pltpu.make_async_copy(rows[b], out_slice(c), semo[b]).wait()

    stage_and_fire(0, 0)

    def pair_body(cc, carry):
        for b in (0, 1):
            c = cc * 2 + b

            @pl.when(c + 1 < NCHUNK)
            def _fire_next():
                @pl.when(c >= 1)
                def _drain_prev():
                    drain_out(c - 1, 1 - b)
                stage_and_fire(c + 1, 1 - b)

            wait_gathers(b)

            def pos_body(p, carry2):
                m = plsc.load_gather(maskv[b], [lax.broadcast(p, (LANES,))])
                for r in range(G):
                    row = p * G + r
                    for h in (0, LANES):
                        rows[b][row, pl.ds(h, LANES)] = (
                            rows[b][row, pl.ds(h, LANES)] * m)
                return carry2
            lax.fori_loop(0, CHUNK, pos_body, 0, unroll=False)

            pltpu.async_copy(rows[b], out_slice(c), semo[b])
        return carry

    lax.fori_loop(0, NCHUNK // 2, pair_body, 0, unroll=False)
    drain_out(NCHUNK - 2, 0)
    drain_out(NCHUNK - 1, 1)


def kernel(feat_matrix, padding, embed_table):
    sel = feat_matrix[:, :, :G].reshape(BL * G // STREAM, STREAM)
    sel = sel.astype(jnp.int32)
    maskf = 1.0 - padding.reshape(-1).astype(jnp.float32)
    out = _feat_embed(sel, maskf, embed_table)
    return out.reshape(B, L, G * D)


# speedup vs baseline: 9.0099x; 1.0337x over previous
import functools

import jax
import jax.numpy as jnp
from jax import lax
from jax.experimental import pallas as pl
from jax.experimental.pallas import tpu as pltpu
from jax.experimental.pallas import tpu_sc as plsc

B = 1024
L = 50
BL = B * L
G = 10
D = 32

NC = 2
NS = 16
NW = NC * NS
LANES = 16

POS_PER_W = BL // NW
CHUNK = 160
NCHUNK = POS_PER_W // CHUNK
STREAM = 80
NGRP = CHUNK * G // STREAM
ROWS = CHUNK * G

_mesh = plsc.VectorSubcoreMesh(
    core_axis_name="c", subcore_axis_name="s", num_cores=NC, num_subcores=NS
)


@functools.partial(
    pl.kernel,
    out_type=jax.ShapeDtypeStruct((BL * G, D), jnp.float32),
    mesh=_mesh,
    compiler_params=pltpu.CompilerParams(
        use_tc_tiling_on_sc=False, needs_layout_passes=False),
    scratch_types=[
        pltpu.VMEM((NGRP, STREAM), jnp.int32),
        pltpu.VMEM((NGRP, STREAM), jnp.int32),
        pltpu.VMEM((CHUNK,), jnp.float32),
        pltpu.VMEM((CHUNK,), jnp.float32),
        pltpu.VMEM((ROWS, D), jnp.float32),
        pltpu.VMEM((ROWS, D), jnp.float32),
        pltpu.SemaphoreType.DMA,
        pltpu.SemaphoreType.DMA,
        pltpu.SemaphoreType.DMA,
        pltpu.SemaphoreType.DMA,
    ],
)
def _feat_embed(sel_hbm, mask_hbm, table_hbm, out_hbm,
                idx0, idx1, mask0, mask1, rows0, rows1,
                semg0, semg1, semo0, semo1):
    idx = (idx0, idx1)
    maskv = (mask0, mask1)
    rows = (rows0, rows1)
    semg = (semg0, semg1)
    semo = (semo0, semo1)

    wid = lax.axis_index("s") * NC + lax.axis_index("c")
    wpos0 = wid * POS_PER_W

    def stage_and_fire(c, b):
        pos0 = pl.multiple_of(wpos0 + c * CHUNK, 8)
        pltpu.sync_copy(sel_hbm.at[pl.ds(pos0 * G // STREAM, NGRP)], idx[b])
        pltpu.sync_copy(mask_hbm.at[pl.ds(pos0, CHUNK)], maskv[b])
        for g in range(NGRP):
            pltpu.async_copy(
                table_hbm.at[idx[b].at[g]],
                rows[b].at[pl.ds(g * STREAM, STREAM)],
                semg[b])

    def wait_gathers(b):
        for g in range(NGRP):
            pltpu.make_async_copy(
                table_hbm.at[idx[b].at[g]],
                rows[b].at[pl.ds(g * STREAM, STREAM)],
                semg[b]).wait()

    def out_slice(c):
        row0 = pl.multiple_of((wpos0 + c * CHUNK) * G, 8)
        return out_hbm.at[pl.ds(row0, ROWS)]

    def drain_out(c, b):
        pltpu.make_async_copy(rows[b], out_slice(c), semo[b]).wait()

    stage_and_fire(0, 0)

    def pair_body(cc, carry):
        for b in (0, 1):
            c = cc * 2 + b

            @pl.when(c + 1 < NCHUNK)
            def _fire_next():
                @pl.when(c >= 1)
                def _drain_prev():
                    drain_out(c - 1, 1 - b)
                stage_and_fire(c + 1, 1 - b)

            wait_gathers(b)

            def pos_body(p, carry2):
                m = plsc.load_gather(maskv[b], [lax.broadcast(p, (LANES,))])
                for r in range(G):
                    row = p * G + r
                    for h in (0, LANES):
                        rows[b][row, pl.ds(h, LANES)] = (
                            rows[b][row, pl.ds(h, LANES)] * m)
                return carry2
            lax.fori_loop(0, CHUNK, pos_body, 0, unroll=False)

            pltpu.async_copy(rows[b], out_slice(c), semo[b])
        return carry

    lax.fori_loop(0, NCHUNK // 2, pair_body, 0, unroll=False)
    drain_out(NCHUNK - 2, 0)
    drain_out(NCHUNK - 1, 1)


def kernel(feat_matrix, padding, embed_table):
    sel = feat_matrix[:, :, :G].reshape(BL * G // STREAM, STREAM)
    sel = sel.astype(jnp.int32)
    maskf = 1.0 - padding.reshape(-1).astype(jnp.float32)
    out = _feat_embed(sel, maskf, embed_table)
    return out.reshape(B, L, G * D)


# speedup vs baseline: 9.3200x; 1.0344x over previous
import functools

import jax
import jax.numpy as jnp
from jax import lax
from jax.experimental import pallas as pl
from jax.experimental.pallas import tpu as pltpu
from jax.experimental.pallas import tpu_sc as plsc

B = 1024
L = 50
BL = B * L
G = 10
D = 32

NC = 2
NS = 16
NW = NC * NS
LANES = 16

POS_PER_W = BL // NW
CHUNK = 160
NCHUNK = POS_PER_W // CHUNK
STREAM = 100
NGRP = CHUNK * G // STREAM
ROWS = CHUNK * G

_mesh = plsc.VectorSubcoreMesh(
    core_axis_name="c", subcore_axis_name="s", num_cores=NC, num_subcores=NS
)


@functools.partial(
    pl.kernel,
    out_type=jax.ShapeDtypeStruct((BL * G, D), jnp.float32),
    mesh=_mesh,
    compiler_params=pltpu.CompilerParams(
        use_tc_tiling_on_sc=False, needs_layout_passes=False),
    scratch_types=[
        pltpu.VMEM((NGRP, STREAM), jnp.int32),
        pltpu.VMEM((NGRP, STREAM), jnp.int32),
        pltpu.VMEM((CHUNK,), jnp.float32),
        pltpu.VMEM((CHUNK,), jnp.float32),
        pltpu.VMEM((ROWS, D), jnp.float32),
        pltpu.VMEM((ROWS, D), jnp.float32),
        pltpu.SemaphoreType.DMA,
        pltpu.SemaphoreType.DMA,
        pltpu.SemaphoreType.DMA,
        pltpu.SemaphoreType.DMA,
    ],
)
def _feat_embed(sel_hbm, mask_hbm, table_hbm, out_hbm,
                idx0, idx1, mask0, mask1, rows0, rows1,
                semg0, semg1, semo0, semo1):
    idx = (idx0, idx1)
    maskv = (mask0, mask1)
    rows = (rows0, rows1)
    semg = (semg0, semg1)
    semo = (semo0, semo1)

    wid = lax.axis_index("s") * NC + lax.axis_index("c")
    wpos0 = wid * POS_PER_W

    def stage_and_fire(c, b):
        pos0 = pl.multiple_of(wpos0 + c * CHUNK, 8)
        pltpu.sync_copy(sel_hbm.at[pl.ds(pos0 * G // STREAM, NGRP)], idx[b])
        pltpu.sync_copy(mask_hbm.at[pl.ds(pos0, CHUNK)], maskv[b])
        for g in range(NGRP):
            pltpu.async_copy(
                table_hbm.at[idx[b].at[g]],
                rows[b].at[pl.ds(g * STREAM, STREAM)],
                semg[b])

    def wait_gathers(b):
        for g in range(NGRP):
            pltpu.make_async_copy(
                table_hbm.at[idx[b].at[g]],
                rows[b].at[pl.ds(g * STREAM, STREAM)],
                semg[b]).wait()

    def out_slice(c):
        row0 = pl.multiple_of((wpos0 + c * CHUNK) * G, 8)
        return out_hbm.at[pl.ds(row0, ROWS)]

    def drain_out(c, b):
        pltpu.make_async_copy(rows[b], out_slice(c), semo[b]).wait()

    stage_and_fire(0, 0)

    def pair_body(cc, carry):
        for b in (0, 1):
            c = cc * 2 + b

            @pl.when(c + 1 < NCHUNK)
            def _fire_next():
                @pl.when(c >= 1)
                def _drain_prev():
                    drain_out(c - 1, 1 - b)
                stage_and_fire(c + 1, 1 - b)

            wait_gathers(b)

            def pos_body(p, carry2):
                m = plsc.load_gather(maskv[b], [lax.broadcast(p, (LANES,))])
                for r in range(G):
                    row = p * G + r
                    for h in (0, LANES):
                        rows[b][row, pl.ds(h, LANES)] = (
                            rows[b][row, pl.ds(h, LANES)] * m)
                return carry2
            lax.fori_loop(0, CHUNK, pos_body, 0, unroll=False)

            pltpu.async_copy(rows[b], out_slice(c), semo[b])
        return carry

    lax.fori_loop(0, NCHUNK // 2, pair_body, 0, unroll=False)
    drain_out(NCHUNK - 2, 0)
    drain_out(NCHUNK - 1, 1)


def kernel(feat_matrix, padding, embed_table):
    sel = feat_matrix[:, :, :G].reshape(BL * G // STREAM, STREAM)
    sel = sel.astype(jnp.int32)
    maskf = 1.0 - padding.reshape(-1).astype(jnp.float32)
    out = _feat_embed(sel, maskf, embed_table)
    return out.reshape(B, L, G * D)
